# baseline (speedup 1.0000x reference)
import functools

import jax
import jax.numpy as jnp
from jax import lax
from jax.experimental import pallas as pl
from jax.experimental.pallas import tpu as pltpu
from jax.experimental.pallas import tpu_sc as plsc

_B = 16384
_NC = 2
_NS = 16
_NW = _NC * _NS
_BPW = _B // _NW
_CH = 128
_NCHUNK = _BPW // _CH


_mesh = plsc.VectorSubcoreMesh(core_axis_name="c", subcore_axis_name="s")

_F32 = functools.partial(jax.ShapeDtypeStruct, dtype=jnp.float32)


@functools.partial(
    pl.kernel,
    out_type=tuple(_F32((_B, 128)) for _ in range(7)),
    mesh=_mesh,
    scratch_types=[
        pltpu.VMEM((_CH,), jnp.int32),
        pltpu.VMEM((_CH,), jnp.int32),
        pltpu.VMEM((_CH, 128), jnp.float32),
        pltpu.VMEM_SHARED((_NS * _CH, 128), jnp.float32),
        pltpu.SemaphoreType.DMA,
    ],
)
def _sc_gather(idx_all, p1, p2, p3a, p3b, spt, dpt, appt,
               o1, o2, o3a, o3b, osp, odp, oapp,
               idxb, ident, tmp, acc, sem):
    sid = lax.axis_index("s")
    wid = sid * _NC + lax.axis_index("c")
    base = wid * _BPW
    arow = sid * _CH

    def mkident(g, carry):
        ident[pl.ds(g * 16, 16)] = lax.iota(jnp.int32, 16) + (arow + g * 16)
        return carry
    lax.fori_loop(0, _CH // 16, mkident, 0)

    groups = ((0, 4, p1, o1), (4, 4, p2, o2), (8, 4, p3a, o3a),
              (12, 2, p3b, o3b))
    singles = ((14, spt, osp), (15, dpt, odp), (16, appt, oapp))

    def chunk(c, carry):
        rb = base + c * _CH
        for (f0, n, tbl, out) in groups:
            for j in range(n):
                pltpu.sync_copy(idx_all.at[pl.ds((f0 + j) * _B + rb, _CH)],
                                idxb)
                pltpu.async_copy(tbl.at[idxb], tmp, sem).wait()
                if j == 0:
                    pltpu.sync_copy(tmp, acc.at[pl.ds(arow, _CH)])
                else:
                    pltpu.sync_copy(tmp, acc.at[ident], add=True)
            pltpu.sync_copy(acc.at[pl.ds(arow, _CH)], out.at[pl.ds(rb, _CH)])
        for (f, tbl, out) in singles:
            pltpu.sync_copy(idx_all.at[pl.ds(f * _B + rb, _CH)], idxb)
            pltpu.async_copy(tbl.at[idxb], tmp, sem).wait()
            pltpu.sync_copy(tmp, out.at[pl.ds(rb, _CH)])
        return carry

    lax.fori_loop(0, _NCHUNK, chunk, 0)


_R = 2048
_GRID = _B // _R

_dot = functools.partial(lax.dot_general,
                         dimension_numbers=(((1,), (1,)), ((), ())),
                         preferred_element_type=jnp.float32)


def _addr_embed(slabs, n, wref, bref):
    es = []
    for s in slabs:
        for q in range(4):
            if len(es) < n:
                es.append(s[:, 32 * q:32 * (q + 1)])
    tot = es[0]
    for e in es[1:]:
        tot = tot + e
    w = wref[...]
    wa = w[0] + w[1] + w[2]
    m = _dot(tot, wa) - _dot(es[0], w[2]) - _dot(es[-1], w[0])
    return m * (1.0 / n) + bref[...]


def _sel_block(slab, low, width, nblk):
    res = slab[:, 0:width]
    for j in range(1, nblk):
        res = jnp.where(low == j, slab[:, width * j:width * (j + 1)], res)
    return res


def _tc_body(o1, o2, o3a, o3b, osp, odp, oapp, spl, dpl, appl, pkt,
             w1, b1, w2, b2, w3, b3, lwT, lb, out):
    e1 = _addr_embed([o1[...]], 4, w1, b1)
    e2 = _addr_embed([o2[...]], 4, w2, b2)
    e3 = _addr_embed([o3a[...], o3b[...]], 6, w3, b3)
    esp = _sel_block(osp[...], spl[...], 32, 4)
    edp = _sel_block(odp[...], dpl[...], 32, 4)
    eapp = _sel_block(oapp[...], appl[...], 16, 8)
    enum = pkt[...] * lwT[...] + lb[...]
    out[...] = jnp.concatenate([e1, e2, e3, esp, edp, eapp, enum], axis=1)


def _tc_final(slabs, spl, dpl, appl, pkt, w1, b1, w2, b2, w3, b3, lwT, lb):
    row = pl.BlockSpec((_R, 128), lambda i: (i, 0))
    col = pl.BlockSpec((_R, 1), lambda i: (i, 0))
    full = lambda s: pl.BlockSpec(s, lambda i: tuple(0 for _ in s))
    return pl.pallas_call(
        _tc_body,
        grid=(_GRID,),
        in_specs=[row] * 7 + [col] * 4 + [full((3, 32, 32)), full((1, 32)),
                                          full((3, 32, 32)), full((1, 32)),
                                          full((3, 32, 32)), full((1, 32)),
                                          full((1, 16)), full((1, 16))],
        out_specs=pl.BlockSpec((_R, 192), lambda i: (i, 0)),
        out_shape=jax.ShapeDtypeStruct((_B, 192), jnp.float32),
    )(*slabs, spl, dpl, appl, pkt, w1, b1, w2, b2, w3, b3, lwT, lb)


def _pack_zero_block(tables, nq):
    cols = []
    for q in range(nq):
        cols.append(jnp.pad(tables[q], ((0, 0), (32 * q, 128 - 32 * (q + 1)))))
    return jnp.concatenate(cols, axis=0)


def kernel(ip_src, ip_dst, mac_src, src_port, dst_port, app_id, pkt_len,
           ip_src_tables, ip_src_conv_w, ip_src_conv_b,
           ip_dst_tables, ip_dst_conv_w, ip_dst_conv_b,
           mac_src_tables, mac_src_conv_w, mac_src_conv_b,
           src_port_table, dst_port_table, app_id_table, lin_w, lin_b):
    ip_src = ip_src.astype(jnp.int32)
    ip_dst = ip_dst.astype(jnp.int32)
    mac_src = mac_src.astype(jnp.int32)
    sp = src_port.astype(jnp.int32)
    dp = dst_port.astype(jnp.int32)
    app = app_id.astype(jnp.int32)

    off4 = jnp.arange(4, dtype=jnp.int32)[:, None] * 256
    off2 = jnp.arange(2, dtype=jnp.int32)[:, None] * 256
    idx_all = jnp.concatenate(
        [(ip_src.T + off4).reshape(-1), (ip_dst.T + off4).reshape(-1),
         (mac_src.T[:4] + off4).reshape(-1), (mac_src.T[4:] + off2).reshape(-1),
         sp >> 2, dp >> 2, app >> 3], axis=0)

    p1 = _pack_zero_block(ip_src_tables, 4)
    p2 = _pack_zero_block(ip_dst_tables, 4)
    p3a = _pack_zero_block(mac_src_tables[:4], 4)
    p3b = _pack_zero_block(mac_src_tables[4:], 2)
    spt = src_port_table.reshape(16384, 128)
    dpt = dst_port_table.reshape(16384, 128)
    appt = app_id_table.reshape(125000, 128)

    slabs = _sc_gather(idx_all, p1, p2, p3a, p3b, spt, dpt, appt)
    return _tc_final(
        slabs,
        (sp & 3).reshape(_B, 1), (dp & 3).reshape(_B, 1),
        (app & 7).reshape(_B, 1),
        pkt_len.reshape(_B, 1).astype(jnp.float32),
        jnp.transpose(ip_src_conv_w, (2, 0, 1)), ip_src_conv_b.reshape(1, 32),
        jnp.transpose(ip_dst_conv_w, (2, 0, 1)), ip_dst_conv_b.reshape(1, 32),
        jnp.transpose(mac_src_conv_w, (2, 0, 1)), mac_src_conv_b.reshape(1, 32),
        lin_w.reshape(1, 16), lin_b.reshape(1, 16))


# speedup vs baseline: 1.0916x; 1.0916x over previous
import functools

import jax
import jax.numpy as jnp
from jax import lax
from jax.experimental import pallas as pl
from jax.experimental.pallas import tpu as pltpu
from jax.experimental.pallas import tpu_sc as plsc

_B = 16384
_NC = 2
_NS = 16
_NW = _NC * _NS
_BPW = _B // _NW
_CH = 256
_NCHUNK = _BPW // _CH
_NF = 17


_mesh = plsc.VectorSubcoreMesh(core_axis_name="c", subcore_axis_name="s")

_F32 = functools.partial(jax.ShapeDtypeStruct, dtype=jnp.float32)


@functools.partial(
    pl.kernel,
    out_type=tuple(_F32((_B, 128)) for _ in range(_NF)),
    mesh=_mesh,
    scratch_types=[
        pltpu.VMEM((_CH,), jnp.int32),
        pltpu.VMEM((_CH,), jnp.int32),
        pltpu.VMEM((_CH,), jnp.int32),
        pltpu.VMEM((_CH, 128), jnp.float32),
        pltpu.VMEM((_CH, 128), jnp.float32),
        pltpu.VMEM((_CH, 128), jnp.float32),
        pltpu.SemaphoreType.DMA,
        pltpu.SemaphoreType.DMA,
        pltpu.SemaphoreType.DMA,
        pltpu.SemaphoreType.DMA,
        pltpu.SemaphoreType.DMA,
        pltpu.SemaphoreType.DMA,
    ],
)
def _sc_gather(idx_all, *rest):
    tables = rest[:_NF]
    outs = rest[_NF:2 * _NF]
    scratch = rest[2 * _NF:]
    idxb = scratch[0:3]
    tmp = scratch[3:6]
    gsem = scratch[6:9]
    wsem = scratch[9:12]

    sid = lax.axis_index("s")
    wid = sid * _NC + lax.axis_index("c")
    base = wid * _BPW

    for c in range(_NCHUNK):
        rb = base + c * _CH
        gh = [None, None, None]
        wh = [None, None, None]
        for f in range(_NF):
            i = f % 3
            if wh[i] is not None:
                wh[i].wait()
                wh[i] = None
            pltpu.sync_copy(idx_all.at[pl.ds(f * _B + rb, _CH)], idxb[i])
            gh[i] = pltpu.async_copy(tables[f].at[idxb[i]], tmp[i], gsem[i])
            if f >= 1:
                j = (f - 1) % 3
                gh[j].wait()
                wh[j] = pltpu.async_copy(tmp[j], outs[f - 1].at[pl.ds(rb, _CH)],
                                         wsem[j])
        j = (_NF - 1) % 3
        gh[j].wait()
        wh[j] = pltpu.async_copy(tmp[j], outs[_NF - 1].at[pl.ds(rb, _CH)],
                                 wsem[j])
        for j in range(3):
            if wh[j] is not None:
                wh[j].wait()


_R = 1024
_GRID = _B // _R

_dot = functools.partial(lax.dot_general,
                         dimension_numbers=(((1,), (1,)), ((), ())),
                         preferred_element_type=jnp.float32)


def _addr_embed(slabs, wref, bref):
    es = [s[:, 0:32] for s in slabs]
    n = len(es)
    tot = es[0]
    for e in es[1:]:
        tot = tot + e
    w = wref[...]
    wa = w[0] + w[1] + w[2]
    m = _dot(tot, wa) - _dot(es[0], w[2]) - _dot(es[-1], w[0])
    return m * (1.0 / n) + bref[...]


def _sel_block(slab, low, width, nblk):
    res = slab[:, 0:width]
    for j in range(1, nblk):
        res = jnp.where(low == j, slab[:, width * j:width * (j + 1)], res)
    return res


def _tc_body(*refs):
    octs = [r[...] for r in refs[:14]]
    (osp, odp, oapp, spl, dpl, appl, pkt,
     w1, b1, w2, b2, w3, b3, lwT, lb, out) = refs[14:]
    e1 = _addr_embed(octs[0:4], w1, b1)
    e2 = _addr_embed(octs[4:8], w2, b2)
    e3 = _addr_embed(octs[8:14], w3, b3)
    esp = _sel_block(osp[...], spl[...], 32, 4)
    edp = _sel_block(odp[...], dpl[...], 32, 4)
    eapp = _sel_block(oapp[...], appl[...], 16, 8)
    enum = pkt[...] * lwT[...] + lb[...]
    out[...] = jnp.concatenate([e1, e2, e3, esp, edp, eapp, enum], axis=1)


def _tc_final(slabs, spl, dpl, appl, pkt, w1, b1, w2, b2, w3, b3, lwT, lb):
    row = pl.BlockSpec((_R, 128), lambda i: (i, 0))
    col = pl.BlockSpec((_R, 1), lambda i: (i, 0))
    full = lambda s: pl.BlockSpec(s, lambda i: tuple(0 for _ in s))
    return pl.pallas_call(
        _tc_body,
        grid=(_GRID,),
        in_specs=[row] * _NF + [col] * 4 + [full((3, 32, 32)), full((1, 32)),
                                            full((3, 32, 32)), full((1, 32)),
                                            full((3, 32, 32)), full((1, 32)),
                                            full((1, 16)), full((1, 16))],
        out_specs=pl.BlockSpec((_R, 192), lambda i: (i, 0)),
        out_shape=jax.ShapeDtypeStruct((_B, 192), jnp.float32),
    )(*slabs, spl, dpl, appl, pkt, w1, b1, w2, b2, w3, b3, lwT, lb)


def kernel(ip_src, ip_dst, mac_src, src_port, dst_port, app_id, pkt_len,
           ip_src_tables, ip_src_conv_w, ip_src_conv_b,
           ip_dst_tables, ip_dst_conv_w, ip_dst_conv_b,
           mac_src_tables, mac_src_conv_w, mac_src_conv_b,
           src_port_table, dst_port_table, app_id_table, lin_w, lin_b):
    ip_src = ip_src.astype(jnp.int32)
    ip_dst = ip_dst.astype(jnp.int32)
    mac_src = mac_src.astype(jnp.int32)
    sp = src_port.astype(jnp.int32)
    dp = dst_port.astype(jnp.int32)
    app = app_id.astype(jnp.int32)

    idx_all = jnp.concatenate(
        [ip_src.T.reshape(-1), ip_dst.T.reshape(-1), mac_src.T.reshape(-1),
         sp >> 2, dp >> 2, app >> 3], axis=0)

    pad = lambda t: jnp.pad(t, ((0, 0), (0, 96)))
    tables = ([pad(ip_src_tables[q]) for q in range(4)]
              + [pad(ip_dst_tables[q]) for q in range(4)]
              + [pad(mac_src_tables[q]) for q in range(6)]
              + [src_port_table.reshape(16384, 128),
                 dst_port_table.reshape(16384, 128),
                 app_id_table.reshape(125000, 128)])

    slabs = _sc_gather(idx_all, *tables)
    return _tc_final(
        slabs,
        (sp & 3).reshape(_B, 1), (dp & 3).reshape(_B, 1),
        (app & 7).reshape(_B, 1),
        pkt_len.reshape(_B, 1).astype(jnp.float32),
        jnp.transpose(ip_src_conv_w, (2, 0, 1)), ip_src_conv_b.reshape(1, 32),
        jnp.transpose(ip_dst_conv_w, (2, 0, 1)), ip_dst_conv_b.reshape(1, 32),
        jnp.transpose(mac_src_conv_w, (2, 0, 1)), mac_src_conv_b.reshape(1, 32),
        lin_w.reshape(1, 16), lin_b.reshape(1, 16))
